# baseline (speedup 1.0000x reference)
import functools
import math

import jax
import jax.numpy as jnp
from jax import lax
from jax.experimental import pallas as pl
from jax.experimental.pallas import tpu as pltpu
from jax.experimental.pallas import tpu_sc as plsc

B = 1024
V = 100000
M = 0.5
COS_M = math.cos(M)
SIN_M = math.sin(M)


_NC = 2
_NS = 16
_NW = _NC * _NS
_BPW = B // _NW


@functools.cache
def _build_sc_gather():
    mesh = plsc.VectorSubcoreMesh(core_axis_name="c", subcore_axis_name="s")

    @functools.partial(
        pl.kernel,
        mesh=mesh,
        out_type=jax.ShapeDtypeStruct((B,), jnp.float32),
        scratch_types=[
            pltpu.VMEM((_BPW,), jnp.int32),
            pltpu.VMEM((_BPW,), jnp.int32),
            pltpu.VMEM((_BPW,), jnp.float32),
            pltpu.SemaphoreType.DMA,
        ],
    )
    def gather_kernel(flat_hbm, tgt_hbm, out_hbm, tgt_v, idx_v, c_v, sem):
        wid = lax.axis_index("s") * _NC + lax.axis_index("c")
        base = wid * _BPW
        pltpu.sync_copy(tgt_hbm.at[pl.ds(base, _BPW)], tgt_v)
        for j in range(_BPW // 16):
            t = tgt_v[pl.ds(j * 16, 16)]
            rows = lax.iota(jnp.int32, 16) + (base + j * 16)
            idx_v[pl.ds(j * 16, 16)] = rows * V + t
        pltpu.async_copy(flat_hbm.at[idx_v], c_v, sem).wait()
        pltpu.sync_copy(c_v, out_hbm.at[pl.ds(base, _BPW)])

    return gather_kernel


_RB = 8
_NR = B // _RB
_CH = 2048
_NFULL = V // _CH
_TAIL0 = _NFULL * _CH
_TAIL_128 = ((V - _TAIL0) // 128) * 128
_REM = V - _TAIL0 - _TAIL_128


def _tc_body(x_ref, c_ref, out_ref):
    i = pl.program_id(0)

    acc = jnp.exp(x_ref[:, 0:_CH])
    for k in range(1, _NFULL):
        acc += jnp.exp(x_ref[:, k * _CH:(k + 1) * _CH])
    rowsum = jnp.sum(acc, axis=1, keepdims=True)
    rowsum += jnp.sum(jnp.exp(x_ref[:, _TAIL0:_TAIL0 + _TAIL_128]),
                      axis=1, keepdims=True)
    rowsum += jnp.sum(jnp.exp(x_ref[:, _TAIL0 + _TAIL_128:V]),
                      axis=1, keepdims=True)

    c = c_ref[...]
    sin_t = jnp.sqrt(jnp.maximum(1.0 - c * c, 0.0))
    new_cos = c * COS_M - sin_t * SIN_M
    stot = rowsum - jnp.exp(c) + jnp.exp(new_cos)
    nll = jnp.log(stot) - new_cos
    partial = jnp.sum(nll) / B

    @pl.when(i == 0)
    def _init():
        out_ref[0, 0] = partial

    @pl.when(i > 0)
    def _accum():
        out_ref[0, 0] += partial


def _tc_loss(inp, c):
    return pl.pallas_call(
        _tc_body,
        grid=(_NR,),
        in_specs=[
            pl.BlockSpec((_RB, V), lambda i: (i, 0)),
            pl.BlockSpec((_RB, 1), lambda i: (i, 0)),
        ],
        out_specs=pl.BlockSpec(memory_space=pltpu.SMEM),
        out_shape=jax.ShapeDtypeStruct((1, 1), jnp.float32),
    )(inp, c)


def kernel(input, target):
    flat = input.reshape(B * V)
    c = _build_sc_gather()(flat, target.astype(jnp.int32))
    out = _tc_loss(input, c.reshape(B, 1))
    return out[0, 0]
